# baseline (speedup 1.0000x reference)
import jax
import jax.numpy as jnp
from jax.experimental import pallas as pl
from jax.experimental.pallas import tpu as pltpu

B = 1024
M = 100000
KD = 16
VD = 16
M_BLK = 4096
MP = 102400
NB = MP // M_BLK


def _norms_body(kt_ref, o_ref):
    ka = kt_ref[0:KD, :].astype(jnp.float32)
    n2 = jnp.sum(ka * ka, axis=0, keepdims=True)
    o_ref[...] = jnp.max(n2, axis=1, keepdims=True)


def _main_body(km2_ref, q_ref, kt_ref, vt_ref, o_ref, qext_ref, acc_ref):
    i = pl.program_id(0)

    @pl.when(i == 0)
    def _():
        q = q_ref[...]
        qn = jnp.sum(q * q, axis=1, keepdims=True)
        u = jnp.sqrt(qn * km2_ref[...])
        qext_ref[:, 0:KD] = q.astype(jnp.bfloat16)
        qext_ref[:, KD : KD + 1] = (-u).astype(jnp.bfloat16)
        acc_ref[...] = jnp.zeros_like(acc_ref)

    s = jnp.dot(qext_ref[...], kt_ref[...], preferred_element_type=jnp.float32)
    p = jnp.exp(s).astype(jnp.bfloat16)
    acc_ref[...] += jax.lax.dot_general(
        p, vt_ref[...], (((1,), (1,)), ((), ())),
        preferred_element_type=jnp.float32,
    )

    @pl.when(i == NB - 1)
    def _():
        o_ref[...] = acc_ref[:, 0:VD] / acc_ref[:, VD : VD + 1]


@jax.jit
def kernel(query, keys, values):
    bias_row = jnp.pad(
        jnp.ones((1, M), jnp.bfloat16), ((0, 0), (0, MP - M)),
        constant_values=64,
    )
    kt_ext = jnp.concatenate(
        [jnp.pad(keys.T.astype(jnp.bfloat16), ((0, 0), (0, MP - M))), bias_row],
        axis=0,
    )
    vt_ext = jnp.concatenate(
        [
            jnp.pad(values.T.astype(jnp.bfloat16), ((0, 0), (0, MP - M))),
            jnp.ones((1, MP), jnp.bfloat16),
        ],
        axis=0,
    )
    km2 = pl.pallas_call(
        _norms_body,
        grid=(1,),
        in_specs=[pl.BlockSpec((KD + 1, MP), lambda t: (0, 0))],
        out_specs=pl.BlockSpec((1, 1), lambda t: (0, 0)),
        out_shape=jax.ShapeDtypeStruct((1, 1), jnp.float32),
    )(kt_ext)
    return pl.pallas_call(
        _main_body,
        grid=(NB,),
        in_specs=[
            pl.BlockSpec((1, 1), lambda i: (0, 0)),
            pl.BlockSpec((B, KD), lambda i: (0, 0)),
            pl.BlockSpec((KD + 1, M_BLK), lambda i: (0, i)),
            pl.BlockSpec((VD + 1, M_BLK), lambda i: (0, i)),
        ],
        out_specs=pl.BlockSpec((B, VD), lambda i: (0, 0)),
        out_shape=jax.ShapeDtypeStruct((B, VD), jnp.float32),
        scratch_shapes=[
            pltpu.VMEM((B, KD + 1), jnp.bfloat16),
            pltpu.VMEM((B, VD + 1), jnp.float32),
        ],
    )(km2, query, kt_ext, vt_ext)


# speedup vs baseline: 1.0059x; 1.0059x over previous
import jax
import jax.numpy as jnp
from jax.experimental import pallas as pl
from jax.experimental.pallas import tpu as pltpu

B = 1024
M = 100000
KD = 16
VD = 16
M_BLK = 5120
MP = 102400
NB = MP // M_BLK


def _norms_body(kt_ref, o_ref):
    ka = kt_ref[0:KD, :].astype(jnp.float32)
    n2 = jnp.sum(ka * ka, axis=0, keepdims=True)
    o_ref[...] = jnp.max(n2, axis=1, keepdims=True)


def _main_body(km2_ref, q_ref, kt_ref, vt_ref, o_ref, qext_ref, acc_ref):
    i = pl.program_id(0)

    @pl.when(i == 0)
    def _():
        q = q_ref[...]
        qn = jnp.sum(q * q, axis=1, keepdims=True)
        u = jnp.sqrt(qn * km2_ref[...])
        qext_ref[:, 0:KD] = q.astype(jnp.bfloat16)
        qext_ref[:, KD : KD + 1] = (-u).astype(jnp.bfloat16)
        acc_ref[...] = jnp.zeros_like(acc_ref)

    s = jnp.dot(qext_ref[...], kt_ref[...], preferred_element_type=jnp.float32)
    p = jnp.exp(s).astype(jnp.bfloat16)
    acc_ref[...] += jax.lax.dot_general(
        p, vt_ref[...], (((1,), (1,)), ((), ())),
        preferred_element_type=jnp.float32,
    )

    @pl.when(i == NB - 1)
    def _():
        o_ref[...] = acc_ref[:, 0:VD] / acc_ref[:, VD : VD + 1]


@jax.jit
def kernel(query, keys, values):
    bias_row = jnp.pad(
        jnp.ones((1, M), jnp.bfloat16), ((0, 0), (0, MP - M)),
        constant_values=64,
    )
    kt_ext = jnp.concatenate(
        [jnp.pad(keys.T.astype(jnp.bfloat16), ((0, 0), (0, MP - M))), bias_row],
        axis=0,
    )
    vt_ext = jnp.concatenate(
        [
            jnp.pad(values.T.astype(jnp.bfloat16), ((0, 0), (0, MP - M))),
            jnp.ones((1, MP), jnp.bfloat16),
        ],
        axis=0,
    )
    km2 = pl.pallas_call(
        _norms_body,
        grid=(1,),
        in_specs=[pl.BlockSpec((KD + 1, MP), lambda t: (0, 0))],
        out_specs=pl.BlockSpec((1, 1), lambda t: (0, 0)),
        out_shape=jax.ShapeDtypeStruct((1, 1), jnp.float32),
    )(kt_ext)
    return pl.pallas_call(
        _main_body,
        grid=(NB,),
        in_specs=[
            pl.BlockSpec((1, 1), lambda i: (0, 0)),
            pl.BlockSpec((B, KD), lambda i: (0, 0)),
            pl.BlockSpec((KD + 1, M_BLK), lambda i: (0, i)),
            pl.BlockSpec((VD + 1, M_BLK), lambda i: (0, i)),
        ],
        out_specs=pl.BlockSpec((B, VD), lambda i: (0, 0)),
        out_shape=jax.ShapeDtypeStruct((B, VD), jnp.float32),
        scratch_shapes=[
            pltpu.VMEM((B, KD + 1), jnp.bfloat16),
            pltpu.VMEM((B, VD + 1), jnp.float32),
        ],
    )(km2, query, kt_ext, vt_ext)
